# baseline (speedup 1.0000x reference)
import functools

import jax
import jax.numpy as jnp
from jax import lax
from jax.experimental import pallas as pl
from jax.experimental.pallas import tpu as pltpu
from jax.experimental.pallas import tpu_sc as plsc

NC = 2
NS = 16
L = 16
NW = NC * NS

B = 16384
F = 32
BPW = B // NW


def _mf_body(fromk_hbm, tok_hbm, table_hbm, wb_hbm, out_hbm,
             idx_a, idx_b, rows_a, rows_b, wb_v, out_v, sem_a, sem_b):
    wid = lax.axis_index("s") * NC + lax.axis_index("c")
    base = wid * BPW

    pltpu.sync_copy(fromk_hbm.at[pl.ds(base, BPW)], idx_a)
    pltpu.sync_copy(tok_hbm.at[pl.ds(base, BPW)], idx_b)
    ca = pltpu.async_copy(table_hbm.at[idx_a], rows_a, sem_a)
    cb = pltpu.async_copy(table_hbm.at[idx_b], rows_b, sem_b)
    pltpu.sync_copy(wb_hbm, wb_v)

    w0 = wb_v[pl.ds(0, L)]
    w1 = wb_v[pl.ds(L, L)]
    bias_v = wb_v[pl.ds(2 * L, L)]
    ca.wait()
    cb.wait()

    lane = lax.iota(jnp.int32, L)
    last = lane == (L - 1)

    def row(i, carry):
        a0 = rows_a[i, pl.ds(0, L)]
        a1 = rows_a[i, pl.ds(L, L)]
        b0 = rows_b[i, pl.ds(0, L)]
        b1 = rows_b[i, pl.ds(L, L)]
        p = a0 * b0 * w0 + a1 * b1 * w1
        s = plsc.cumsum(p) + bias_v
        plsc.store_scatter(out_v, [lane * 0 + i], s, mask=last)
        return carry

    lax.fori_loop(0, BPW, row, 0, unroll=8)
    pltpu.sync_copy(out_v, out_hbm.at[pl.ds(base, BPW)])


@functools.partial(
    pl.kernel,
    out_type=jax.ShapeDtypeStruct((B,), jnp.float32),
    mesh=plsc.VectorSubcoreMesh(core_axis_name="c", subcore_axis_name="s",
                                num_cores=NC, num_subcores=NS),
    scratch_types=[
        pltpu.VMEM((BPW,), jnp.int32),
        pltpu.VMEM((BPW,), jnp.int32),
        pltpu.VMEM((BPW, F), jnp.float32),
        pltpu.VMEM((BPW, F), jnp.float32),
        pltpu.VMEM((3 * L,), jnp.float32),
        pltpu.VMEM((BPW,), jnp.float32),
        pltpu.SemaphoreType.DMA,
        pltpu.SemaphoreType.DMA,
    ],
    compiler_params=pltpu.CompilerParams(needs_layout_passes=False,
                                         use_tc_tiling_on_sc=False),
)
def _mf_kernel(*refs):
    _mf_body(*refs)


def kernel(fromk, tok, embed_k_GMF, predict_W, predict_b):
    wb = jnp.concatenate([predict_W.reshape(-1),
                          jnp.broadcast_to(predict_b, (L,))])
    return _mf_kernel(fromk.astype(jnp.int32), tok.astype(jnp.int32),
                      embed_k_GMF, wb)
